# baseline (speedup 1.0000x reference)
import functools

import jax
import jax.numpy as jnp
from jax import lax
from jax.experimental import pallas as pl
from jax.experimental.pallas import tpu as pltpu
from jax.experimental.pallas import tpu_sc as plsc

_G = 8
_BN = 1000
_CH = 1024


def _acc_rows(N):
    return ((N + 1 + 127) // 128) * 128


def _make_degcount(N, Epad):
    acc_rows = _acc_rows(N)
    zr = acc_rows // 16
    rows_per_tile = Epad // 32 // 128
    mesh = plsc.VectorSubcoreMesh(core_axis_name="c", subcore_axis_name="s")

    @functools.partial(
        pl.kernel, mesh=mesh,
        out_type=jax.ShapeDtypeStruct((2, N, 128), jnp.float32),
        scratch_types=[
            pltpu.VMEM((8, 128), jnp.int32),
            pltpu.VMEM((128, 128), jnp.float32),
            pltpu.VMEM_SHARED((acc_rows, 128), jnp.float32),
            pltpu.SemaphoreType.DMA,
        ],
    )
    def degcount(dst_hbm, z16_hbm, ones_hbm, out_hbm, didx, ones_v, acc, sem):
        c = lax.axis_index("c")
        s = lax.axis_index("s")
        pltpu.sync_copy(ones_hbm, ones_v)
        pltpu.sync_copy(z16_hbm, acc.at[pl.ds(s * zr, zr)])
        plsc.subcore_barrier()

        row0 = (c * 16 + s) * rows_per_tile

        def chunk(i, carry):
            pltpu.sync_copy(dst_hbm.at[pl.ds(row0 + i * 8, 8)], didx)
            cps = [pltpu.async_copy(ones_v, acc.at[didx.at[j]], sem,
                                    add=True)
                   for j in range(8)]
            for cp in cps:
                cp.wait()
            return carry

        lax.fori_loop(0, rows_per_tile // 8, chunk, 0)
        plsc.subcore_barrier()
        npt = N // 10

        @pl.when(s < 10)
        def _():
            pltpu.sync_copy(acc.at[pl.ds(s * npt, npt)],
                            out_hbm.at[c, pl.ds(s * npt, npt)])

    return degcount


def _make_propagate(N, Epad, Dh):
    acc_rows = _acc_rows(N)
    ept = Epad // 16
    ir = 16
    nck = ept // 128 // ir
    mesh = plsc.VectorSubcoreMesh(core_axis_name="c", subcore_axis_name="s")

    @functools.partial(
        pl.kernel, mesh=mesh,
        out_type=jax.ShapeDtypeStruct((2 * N, Dh), jnp.float32),
        scratch_types=[
            pltpu.VMEM((ir, 128), jnp.int32),
            pltpu.VMEM((ir, 128), jnp.int32),
            pltpu.VMEM((ir, 128), jnp.int32),
            pltpu.VMEM((ir, 128), jnp.int32),
            pltpu.VMEM((256, Dh), jnp.float32),
            pltpu.VMEM_SHARED((acc_rows, Dh), jnp.float32),
            pltpu.SemaphoreType.DMA,
            pltpu.SemaphoreType.DMA,
            pltpu.SemaphoreType.DMA,
            pltpu.SemaphoreType.DMA,
            pltpu.SemaphoreType.DMA,
        ],
    )
    def propagate(table_hbm, src_hbm, dst_hbm, out_hbm,
                  sidx0, didx0, sidx1, didx1, rows, acc,
                  sg0, sg1, ss0, ss1, si):
        c = lax.axis_index("c")
        s = lax.axis_index("s")
        dst_row0 = s * (ept // 128)
        src_row0 = c * (Epad // 128) + s * (ept // 128)
        npt = N // 10

        @pl.when(s < 10)
        def _():
            pltpu.sync_copy(table_hbm.at[pl.ds(c * N + s * npt, npt)],
                            acc.at[pl.ds(s * npt, npt)])

        plsc.subcore_barrier()

        sg = (sg0, sg1)
        ss = (ss0, ss1)
        bufs = ((sidx0, didx0), (sidx1, didx1))

        def load_idx(cc):
            sb, db = bufs[cc % 2]
            h1 = pltpu.async_copy(src_hbm.at[pl.ds(src_row0 + cc * ir, ir)],
                                  sb, si)
            h2 = pltpu.async_copy(dst_hbm.at[pl.ds(dst_row0 + cc * ir, ir)],
                                  db, si)
            return (h1, h2)

        pend = load_idx(0)
        for cc in range(nck):
            sb, db = bufs[cc % 2]
            pend[0].wait()
            pend[1].wait()
            if cc + 1 < nck:
                pend = load_idx(cc + 1)

            def gth(k, sl, sb=sb):
                a = pltpu.async_copy(table_hbm.at[sb.at[k].at[pl.ds(0, 64)]],
                                     rows.at[pl.ds(sl * 128, 64)], sg[sl])
                b = pltpu.async_copy(table_hbm.at[sb.at[k].at[pl.ds(64, 64)]],
                                     rows.at[pl.ds(sl * 128 + 64, 64)],
                                     sg[sl])
                return (a, b)

            def sct(k, sl, db=db):
                return pltpu.async_copy(rows.at[pl.ds(sl * 128, 128)],
                                        acc.at[db.at[k]], ss[sl], add=True)

            def gwait(g):
                g[0].wait()
                g[1].wait()

            @pl.loop(0, ir, step=4)
            def _(k):
                g0 = gth(k, 0)
                g1 = gth(k + 1, 1)
                gwait(g0)
                s0 = sct(k, 0)
                gwait(g1)
                s1 = sct(k + 1, 1)
                s0.wait()
                g2 = gth(k + 2, 0)
                gwait(g2)
                s2 = sct(k + 2, 0)
                s1.wait()
                g3 = gth(k + 3, 1)
                gwait(g3)
                s3 = sct(k + 3, 1)
                s2.wait()
                s3.wait()

        plsc.subcore_barrier()

        @pl.when(s < 10)
        def _():
            pltpu.sync_copy(acc.at[pl.ds(s * npt, npt)],
                            out_hbm.at[pl.ds(c * N + s * npt, npt)])

    return propagate


def _matmul_body(x_ref, w_ref, o_ref):
    o_ref[...] = jnp.dot(x_ref[...], w_ref[0],
                         preferred_element_type=jnp.float32)


def _scale_dinv_body(hw_ref, a_ref, b_ref, t_ref, d_ref):
    deg = 1.0 + a_ref[0, :, :1] + b_ref[0, :, :1]
    dv = lax.rsqrt(deg)
    t_ref[...] = hw_ref[...] * dv
    d_ref[...] = jnp.broadcast_to(dv, d_ref.shape)


def _layer2_body(s0, s1, dinv_ref, b1_ref, w2_ref, o_ref):
    h = jnp.concatenate([s0[...], s1[...]], axis=1)
    a = jnp.maximum(h * dinv_ref[:, :1] + b1_ref[...], 0.0)
    t = jnp.dot(a, w2_ref[0], preferred_element_type=jnp.float32)
    o_ref[...] = t * dinv_ref[:, :1]


def _make_pool_body(nb):
    def pool_body(s0, s1, dinv_ref, b2_ref, oh_ref,
                  o_ref, sums_ref, cnts_ref):
        i = pl.program_id(0)
        h = jnp.concatenate([s0[...], s1[...]], axis=1)
        out2 = h * dinv_ref[:, :1] + b2_ref[...]
        oh = oh_ref[...]
        ps = jax.lax.dot_general(oh, out2, (((0,), (0,)), ((), ())),
                                 preferred_element_type=jnp.float32)
        pc = jax.lax.dot_general(oh, jnp.ones_like(out2),
                                 (((0,), (0,)), ((), ())),
                                 preferred_element_type=jnp.float32)

        @pl.when(i == 0)
        def _():
            sums_ref[...] = ps
            cnts_ref[...] = pc

        @pl.when(i > 0)
        def _():
            sums_ref[...] += ps
            cnts_ref[...] += pc

        @pl.when(i == nb - 1)
        def _():
            o_ref[...] = sums_ref[...] / jnp.maximum(cnts_ref[...], 1.0)

    return pool_body


def kernel(x, edge_index, batch, W1, b1, W2, b2):
    N, D = x.shape
    E = edge_index.shape[1]
    Dh = D // 2
    nb = N // _BN

    src = edge_index[0]
    dst = edge_index[1]
    Epad = ((E + 16 * _CH - 1) // (16 * _CH)) * (16 * _CH)
    pad = Epad - E
    src_a = jnp.concatenate([src, jnp.zeros((pad,), jnp.int32)])
    src2 = jnp.concatenate(
        [src_a + q * N for q in range(2)]).reshape(2 * Epad // 128, 128)
    dst_p = jnp.concatenate(
        [dst, jnp.full((pad,), N, jnp.int32)]).reshape(Epad // 128, 128)

    zr = _acc_rows(N) // 16
    zeros_h = jnp.zeros((zr, Dh), jnp.float32)
    ones128 = jnp.ones((128, 128), jnp.float32)
    onehot = (batch[:, None] == jnp.arange(_G, dtype=batch.dtype)[None, :]
              ).astype(jnp.float32)
    b1_2d = b1.reshape(1, D)
    b2_2d = b2.reshape(1, D)
    W1s = W1.reshape(D, 2, Dh).transpose(1, 0, 2)
    W2s = W2.reshape(D, 2, Dh).transpose(1, 0, 2)

    degcount = _make_degcount(N, Epad)
    propagate = _make_propagate(N, Epad, Dh)

    hw = pl.pallas_call(
        _matmul_body,
        grid=(nb, 2),
        in_specs=[
            pl.BlockSpec((_BN, D), lambda i, j: (i, 0)),
            pl.BlockSpec((1, D, Dh), lambda i, j: (j, 0, 0)),
        ],
        out_specs=pl.BlockSpec((_BN, Dh), lambda i, j: (j * nb + i, 0)),
        out_shape=jax.ShapeDtypeStruct((2 * N, Dh), jnp.float32),
    )(x, W1s)

    cnt = degcount(dst_p, zeros_h, ones128)

    table1, dinv = pl.pallas_call(
        _scale_dinv_body,
        grid=(nb, 2),
        in_specs=[
            pl.BlockSpec((_BN, Dh), lambda i, j: (j * nb + i, 0)),
            pl.BlockSpec((1, _BN, 128), lambda i, j: (0, i, 0)),
            pl.BlockSpec((1, _BN, 128), lambda i, j: (1, i, 0)),
        ],
        out_specs=[
            pl.BlockSpec((_BN, Dh), lambda i, j: (j * nb + i, 0)),
            pl.BlockSpec((_BN, 128), lambda i, j: (i, 0)),
        ],
        out_shape=[
            jax.ShapeDtypeStruct((2 * N, Dh), jnp.float32),
            jax.ShapeDtypeStruct((N, 128), jnp.float32),
        ],
    )(hw, cnt, cnt)

    scat1 = propagate(table1, src2, dst_p)

    def _slab_spec(q):
        return pl.BlockSpec((_BN, Dh), lambda i, j, q=q: (q * nb + i, 0))

    table2 = pl.pallas_call(
        _layer2_body,
        grid=(nb, 2),
        in_specs=(
            [_slab_spec(q) for q in range(2)]
            + [
                pl.BlockSpec((_BN, 128), lambda i, j: (i, 0)),
                pl.BlockSpec((1, D), lambda i, j: (0, 0)),
                pl.BlockSpec((1, D, Dh), lambda i, j: (j, 0, 0)),
            ]
        ),
        out_specs=pl.BlockSpec((_BN, Dh), lambda i, j: (j * nb + i, 0)),
        out_shape=jax.ShapeDtypeStruct((2 * N, Dh), jnp.float32),
    )(scat1, scat1, dinv, b1_2d, W2s)

    scat2 = propagate(table2, src2, dst_p)

    def _slab_spec1(q):
        return pl.BlockSpec((_BN, Dh), lambda i, q=q: (q * nb + i, 0))

    graph_emb = pl.pallas_call(
        _make_pool_body(nb),
        grid=(nb,),
        in_specs=(
            [_slab_spec1(q) for q in range(2)]
            + [
                pl.BlockSpec((_BN, 128), lambda i: (i, 0)),
                pl.BlockSpec((1, D), lambda i: (0, 0)),
                pl.BlockSpec((_BN, _G), lambda i: (i, 0)),
            ]
        ),
        out_specs=pl.BlockSpec((_G, D), lambda i: (0, 0)),
        out_shape=jax.ShapeDtypeStruct((_G, D), jnp.float32),
        scratch_shapes=[
            pltpu.VMEM((_G, D), jnp.float32),
            pltpu.VMEM((_G, D), jnp.float32),
        ],
    )(scat2, scat2, dinv, b2_2d, onehot)

    return graph_emb


# speedup vs baseline: 1.0017x; 1.0017x over previous
import functools

import jax
import jax.numpy as jnp
from jax import lax
from jax.experimental import pallas as pl
from jax.experimental.pallas import tpu as pltpu
from jax.experimental.pallas import tpu_sc as plsc

_G = 8
_BN = 1000
_CH = 1024


def _acc_rows(N):
    return ((N + 1 + 127) // 128) * 128


def _make_degcount(N, Epad):
    acc_rows = _acc_rows(N)
    zr = acc_rows // 16
    rows_per_tile = Epad // 32 // 128
    mesh = plsc.VectorSubcoreMesh(core_axis_name="c", subcore_axis_name="s")

    @functools.partial(
        pl.kernel, mesh=mesh,
        out_type=jax.ShapeDtypeStruct((2, N, 128), jnp.float32),
        scratch_types=[
            pltpu.VMEM((8, 128), jnp.int32),
            pltpu.VMEM((128, 128), jnp.float32),
            pltpu.VMEM_SHARED((acc_rows, 128), jnp.float32),
            pltpu.SemaphoreType.DMA,
        ],
    )
    def degcount(dst_hbm, z16_hbm, ones_hbm, out_hbm, didx, ones_v, acc, sem):
        c = lax.axis_index("c")
        s = lax.axis_index("s")
        pltpu.sync_copy(ones_hbm, ones_v)
        pltpu.sync_copy(z16_hbm, acc.at[pl.ds(s * zr, zr)])
        plsc.subcore_barrier()

        row0 = (c * 16 + s) * rows_per_tile

        def chunk(i, carry):
            pltpu.sync_copy(dst_hbm.at[pl.ds(row0 + i * 8, 8)], didx)
            cps = [pltpu.async_copy(ones_v, acc.at[didx.at[j]], sem,
                                    add=True)
                   for j in range(8)]
            for cp in cps:
                cp.wait()
            return carry

        lax.fori_loop(0, rows_per_tile // 8, chunk, 0)
        plsc.subcore_barrier()
        npt = N // 10

        @pl.when(s < 10)
        def _():
            pltpu.sync_copy(acc.at[pl.ds(s * npt, npt)],
                            out_hbm.at[c, pl.ds(s * npt, npt)])

    return degcount


def _make_propagate(N, Epad, Dh):
    acc_rows = _acc_rows(N)
    ept = Epad // 16
    ir = 16
    nck = ept // 128 // ir
    mesh = plsc.VectorSubcoreMesh(core_axis_name="c", subcore_axis_name="s")

    @functools.partial(
        pl.kernel, mesh=mesh,
        out_type=jax.ShapeDtypeStruct((2 * N, Dh), jnp.float32),
        scratch_types=[
            pltpu.VMEM((ir, 128), jnp.int32),
            pltpu.VMEM((ir, 128), jnp.int32),
            pltpu.VMEM((ir, 128), jnp.int32),
            pltpu.VMEM((ir, 128), jnp.int32),
            pltpu.VMEM((256, Dh), jnp.float32),
            pltpu.VMEM_SHARED((acc_rows, Dh), jnp.float32),
            pltpu.SemaphoreType.DMA,
            pltpu.SemaphoreType.DMA,
            pltpu.SemaphoreType.DMA,
            pltpu.SemaphoreType.DMA,
            pltpu.SemaphoreType.DMA,
        ],
    )
    def propagate(table_hbm, src_hbm, dst_hbm, out_hbm,
                  sidx0, didx0, sidx1, didx1, rows, acc,
                  sg0, sg1, ss0, ss1, si):
        c = lax.axis_index("c")
        s = lax.axis_index("s")
        dst_row0 = s * (ept // 128)
        src_row0 = c * (Epad // 128) + s * (ept // 128)
        npt = N // 10

        @pl.when(s < 15)
        def _():
            pltpu.sync_copy(table_hbm.at[pl.ds(c * N + s * 632, 632)],
                            acc.at[pl.ds(s * 632, 632)])

        @pl.when(s == 15)
        def _():
            pltpu.sync_copy(table_hbm.at[pl.ds(c * N + 15 * 632, N - 15 * 632)],
                            acc.at[pl.ds(15 * 632, N - 15 * 632)])

        plsc.subcore_barrier()

        sg = (sg0, sg1)
        ss = (ss0, ss1)
        bufs = ((sidx0, didx0), (sidx1, didx1))

        def load_idx(cc):
            sb, db = bufs[cc % 2]
            h1 = pltpu.async_copy(src_hbm.at[pl.ds(src_row0 + cc * ir, ir)],
                                  sb, si)
            h2 = pltpu.async_copy(dst_hbm.at[pl.ds(dst_row0 + cc * ir, ir)],
                                  db, si)
            return (h1, h2)

        pend = load_idx(0)
        for cc in range(nck):
            sb, db = bufs[cc % 2]
            pend[0].wait()
            pend[1].wait()
            if cc + 1 < nck:
                pend = load_idx(cc + 1)

            def gth(k, sl, sb=sb):
                a = pltpu.async_copy(table_hbm.at[sb.at[k].at[pl.ds(0, 64)]],
                                     rows.at[pl.ds(sl * 128, 64)], sg[sl])
                b = pltpu.async_copy(table_hbm.at[sb.at[k].at[pl.ds(64, 64)]],
                                     rows.at[pl.ds(sl * 128 + 64, 64)],
                                     sg[sl])
                return (a, b)

            def sct(k, sl, db=db):
                return pltpu.async_copy(rows.at[pl.ds(sl * 128, 128)],
                                        acc.at[db.at[k]], ss[sl], add=True)

            def gwait(g):
                g[0].wait()
                g[1].wait()

            @pl.loop(0, ir, step=4)
            def _(k):
                g0 = gth(k, 0)
                g1 = gth(k + 1, 1)
                gwait(g0)
                s0 = sct(k, 0)
                gwait(g1)
                s1 = sct(k + 1, 1)
                s0.wait()
                g2 = gth(k + 2, 0)
                gwait(g2)
                s2 = sct(k + 2, 0)
                s1.wait()
                g3 = gth(k + 3, 1)
                gwait(g3)
                s3 = sct(k + 3, 1)
                s2.wait()
                s3.wait()

        plsc.subcore_barrier()

        @pl.when(s < 10)
        def _():
            pltpu.sync_copy(acc.at[pl.ds(s * npt, npt)],
                            out_hbm.at[pl.ds(c * N + s * npt, npt)])

    return propagate


def _matmul_body(x_ref, w_ref, o_ref):
    o_ref[...] = jnp.dot(x_ref[...], w_ref[0],
                         preferred_element_type=jnp.float32)


def _scale_dinv_body(hw_ref, a_ref, b_ref, t_ref, d_ref):
    deg = 1.0 + a_ref[0, :, :1] + b_ref[0, :, :1]
    dv = lax.rsqrt(deg)
    t_ref[...] = hw_ref[...] * dv
    d_ref[...] = jnp.broadcast_to(dv, d_ref.shape)


def _layer2_body(s0, s1, dinv_ref, b1_ref, w2_ref, o_ref):
    h = jnp.concatenate([s0[...], s1[...]], axis=1)
    a = jnp.maximum(h * dinv_ref[:, :1] + b1_ref[...], 0.0)
    t = jnp.dot(a, w2_ref[0], preferred_element_type=jnp.float32)
    o_ref[...] = t * dinv_ref[:, :1]


def _make_pool_body(nb):
    def pool_body(s0, s1, dinv_ref, b2_ref, oh_ref,
                  o_ref, sums_ref, cnts_ref):
        i = pl.program_id(0)
        h = jnp.concatenate([s0[...], s1[...]], axis=1)
        out2 = h * dinv_ref[:, :1] + b2_ref[...]
        oh = oh_ref[...]
        ps = jax.lax.dot_general(oh, out2, (((0,), (0,)), ((), ())),
                                 preferred_element_type=jnp.float32)
        pc = jax.lax.dot_general(oh, jnp.ones_like(out2),
                                 (((0,), (0,)), ((), ())),
                                 preferred_element_type=jnp.float32)

        @pl.when(i == 0)
        def _():
            sums_ref[...] = ps
            cnts_ref[...] = pc

        @pl.when(i > 0)
        def _():
            sums_ref[...] += ps
            cnts_ref[...] += pc

        @pl.when(i == nb - 1)
        def _():
            o_ref[...] = sums_ref[...] / jnp.maximum(cnts_ref[...], 1.0)

    return pool_body


def kernel(x, edge_index, batch, W1, b1, W2, b2):
    N, D = x.shape
    E = edge_index.shape[1]
    Dh = D // 2
    nb = N // _BN

    src = edge_index[0]
    dst = edge_index[1]
    Epad = ((E + 16 * _CH - 1) // (16 * _CH)) * (16 * _CH)
    pad = Epad - E
    src_a = jnp.concatenate([src, jnp.zeros((pad,), jnp.int32)])
    src2 = jnp.concatenate(
        [src_a + q * N for q in range(2)]).reshape(2 * Epad // 128, 128)
    dst_p = jnp.concatenate(
        [dst, jnp.full((pad,), N, jnp.int32)]).reshape(Epad // 128, 128)

    zr = _acc_rows(N) // 16
    zeros_h = jnp.zeros((zr, Dh), jnp.float32)
    ones128 = jnp.ones((128, 128), jnp.float32)
    onehot = (batch[:, None] == jnp.arange(_G, dtype=batch.dtype)[None, :]
              ).astype(jnp.float32)
    b1_2d = b1.reshape(1, D)
    b2_2d = b2.reshape(1, D)
    W1s = W1.reshape(D, 2, Dh).transpose(1, 0, 2)
    W2s = W2.reshape(D, 2, Dh).transpose(1, 0, 2)

    degcount = _make_degcount(N, Epad)
    propagate = _make_propagate(N, Epad, Dh)

    hw = pl.pallas_call(
        _matmul_body,
        grid=(nb, 2),
        in_specs=[
            pl.BlockSpec((_BN, D), lambda i, j: (i, 0)),
            pl.BlockSpec((1, D, Dh), lambda i, j: (j, 0, 0)),
        ],
        out_specs=pl.BlockSpec((_BN, Dh), lambda i, j: (j * nb + i, 0)),
        out_shape=jax.ShapeDtypeStruct((2 * N, Dh), jnp.float32),
    )(x, W1s)

    cnt = degcount(dst_p, zeros_h, ones128)

    table1, dinv = pl.pallas_call(
        _scale_dinv_body,
        grid=(nb, 2),
        in_specs=[
            pl.BlockSpec((_BN, Dh), lambda i, j: (j * nb + i, 0)),
            pl.BlockSpec((1, _BN, 128), lambda i, j: (0, i, 0)),
            pl.BlockSpec((1, _BN, 128), lambda i, j: (1, i, 0)),
        ],
        out_specs=[
            pl.BlockSpec((_BN, Dh), lambda i, j: (j * nb + i, 0)),
            pl.BlockSpec((_BN, 128), lambda i, j: (i, 0)),
        ],
        out_shape=[
            jax.ShapeDtypeStruct((2 * N, Dh), jnp.float32),
            jax.ShapeDtypeStruct((N, 128), jnp.float32),
        ],
    )(hw, cnt, cnt)

    scat1 = propagate(table1, src2, dst_p)

    def _slab_spec(q):
        return pl.BlockSpec((_BN, Dh), lambda i, j, q=q: (q * nb + i, 0))

    table2 = pl.pallas_call(
        _layer2_body,
        grid=(nb, 2),
        in_specs=(
            [_slab_spec(q) for q in range(2)]
            + [
                pl.BlockSpec((_BN, 128), lambda i, j: (i, 0)),
                pl.BlockSpec((1, D), lambda i, j: (0, 0)),
                pl.BlockSpec((1, D, Dh), lambda i, j: (j, 0, 0)),
            ]
        ),
        out_specs=pl.BlockSpec((_BN, Dh), lambda i, j: (j * nb + i, 0)),
        out_shape=jax.ShapeDtypeStruct((2 * N, Dh), jnp.float32),
    )(scat1, scat1, dinv, b1_2d, W2s)

    scat2 = propagate(table2, src2, dst_p)

    def _slab_spec1(q):
        return pl.BlockSpec((_BN, Dh), lambda i, q=q: (q * nb + i, 0))

    graph_emb = pl.pallas_call(
        _make_pool_body(nb),
        grid=(nb,),
        in_specs=(
            [_slab_spec1(q) for q in range(2)]
            + [
                pl.BlockSpec((_BN, 128), lambda i: (i, 0)),
                pl.BlockSpec((1, D), lambda i: (0, 0)),
                pl.BlockSpec((_BN, _G), lambda i: (i, 0)),
            ]
        ),
        out_specs=pl.BlockSpec((_G, D), lambda i: (0, 0)),
        out_shape=jax.ShapeDtypeStruct((_G, D), jnp.float32),
        scratch_shapes=[
            pltpu.VMEM((_G, D), jnp.float32),
            pltpu.VMEM((_G, D), jnp.float32),
        ],
    )(scat2, scat2, dinv, b2_2d, onehot)

    return graph_emb


# speedup vs baseline: 1.0025x; 1.0008x over previous
import functools

import jax
import jax.numpy as jnp
from jax import lax
from jax.experimental import pallas as pl
from jax.experimental.pallas import tpu as pltpu
from jax.experimental.pallas import tpu_sc as plsc

_G = 8
_BN = 1000
_CH = 1024


def _acc_rows(N):
    return ((N + 1 + 127) // 128) * 128


def _make_degcount(N, Epad):
    acc_rows = _acc_rows(N)
    zr = acc_rows // 16
    rows_per_tile = Epad // 32 // 128
    mesh = plsc.VectorSubcoreMesh(core_axis_name="c", subcore_axis_name="s")

    @functools.partial(
        pl.kernel, mesh=mesh,
        out_type=jax.ShapeDtypeStruct((2, N, 128), jnp.float32),
        scratch_types=[
            pltpu.VMEM((8, 128), jnp.int32),
            pltpu.VMEM((128, 128), jnp.float32),
            pltpu.VMEM_SHARED((acc_rows, 128), jnp.float32),
            pltpu.SemaphoreType.DMA,
        ],
    )
    def degcount(dst_hbm, z16_hbm, ones_hbm, out_hbm, didx, ones_v, acc, sem):
        c = lax.axis_index("c")
        s = lax.axis_index("s")
        pltpu.sync_copy(ones_hbm, ones_v)
        pltpu.sync_copy(z16_hbm, acc.at[pl.ds(s * zr, zr)])
        plsc.subcore_barrier()

        row0 = (c * 16 + s) * rows_per_tile

        def chunk(i, carry):
            pltpu.sync_copy(dst_hbm.at[pl.ds(row0 + i * 8, 8)], didx)
            cps = [pltpu.async_copy(ones_v, acc.at[didx.at[j]], sem,
                                    add=True)
                   for j in range(8)]
            for cp in cps:
                cp.wait()
            return carry

        lax.fori_loop(0, rows_per_tile // 8, chunk, 0)
        plsc.subcore_barrier()
        npt = N // 10

        @pl.when(s < 10)
        def _():
            pltpu.sync_copy(acc.at[pl.ds(s * npt, npt)],
                            out_hbm.at[c, pl.ds(s * npt, npt)])

    return degcount


def _make_propagate(N, Epad, Dh):
    acc_rows = _acc_rows(N)
    ept = Epad // 16
    ir = 40
    nck = ept // 128 // ir
    mesh = plsc.VectorSubcoreMesh(core_axis_name="c", subcore_axis_name="s")

    @functools.partial(
        pl.kernel, mesh=mesh,
        out_type=jax.ShapeDtypeStruct((2 * N, Dh), jnp.float32),
        scratch_types=[
            pltpu.VMEM((ir, 128), jnp.int32),
            pltpu.VMEM((ir, 128), jnp.int32),
            pltpu.VMEM((256, Dh), jnp.float32),
            pltpu.VMEM_SHARED((acc_rows, Dh), jnp.float32),
            pltpu.SemaphoreType.DMA,
            pltpu.SemaphoreType.DMA,
            pltpu.SemaphoreType.DMA,
            pltpu.SemaphoreType.DMA,
        ],
    )
    def propagate(table_hbm, src_hbm, dst_hbm, out_hbm,
                  sidx, didx, rows, acc, sg0, sg1, ss0, ss1):
        c = lax.axis_index("c")
        s = lax.axis_index("s")
        dst_row0 = s * (ept // 128)
        src_row0 = c * (Epad // 128) + s * (ept // 128)
        npt = N // 10

        @pl.when(s < 15)
        def _():
            pltpu.sync_copy(table_hbm.at[pl.ds(c * N + s * 632, 632)],
                            acc.at[pl.ds(s * 632, 632)])

        @pl.when(s == 15)
        def _():
            pltpu.sync_copy(table_hbm.at[pl.ds(c * N + 15 * 632, N - 15 * 632)],
                            acc.at[pl.ds(15 * 632, N - 15 * 632)])

        plsc.subcore_barrier()

        sg = (sg0, sg1)
        ss = (ss0, ss1)

        for cc in range(nck):
            sb, db = sidx, didx
            pltpu.sync_copy(src_hbm.at[pl.ds(src_row0 + cc * ir, ir)], sb)
            pltpu.sync_copy(dst_hbm.at[pl.ds(dst_row0 + cc * ir, ir)], db)

            def gth(k, sl, sb=sb):
                a = pltpu.async_copy(table_hbm.at[sb.at[k].at[pl.ds(0, 64)]],
                                     rows.at[pl.ds(sl * 128, 64)], sg[sl])
                b = pltpu.async_copy(table_hbm.at[sb.at[k].at[pl.ds(64, 64)]],
                                     rows.at[pl.ds(sl * 128 + 64, 64)],
                                     sg[sl])
                return (a, b)

            def sct(k, sl, db=db):
                return pltpu.async_copy(rows.at[pl.ds(sl * 128, 128)],
                                        acc.at[db.at[k]], ss[sl], add=True)

            def gwait(g):
                g[0].wait()
                g[1].wait()

            @pl.loop(0, ir, step=4)
            def _(k):
                g0 = gth(k, 0)
                g1 = gth(k + 1, 1)
                gwait(g0)
                s0 = sct(k, 0)
                gwait(g1)
                s1 = sct(k + 1, 1)
                s0.wait()
                g2 = gth(k + 2, 0)
                gwait(g2)
                s2 = sct(k + 2, 0)
                s1.wait()
                g3 = gth(k + 3, 1)
                gwait(g3)
                s3 = sct(k + 3, 1)
                s2.wait()
                s3.wait()

        plsc.subcore_barrier()

        @pl.when(s < 10)
        def _():
            pltpu.sync_copy(acc.at[pl.ds(s * npt, npt)],
                            out_hbm.at[pl.ds(c * N + s * npt, npt)])

    return propagate


def _matmul_body(x_ref, w_ref, o_ref):
    o_ref[...] = jnp.dot(x_ref[...], w_ref[0],
                         preferred_element_type=jnp.float32)


def _scale_dinv_body(hw_ref, a_ref, b_ref, t_ref, d_ref):
    deg = 1.0 + a_ref[0, :, :1] + b_ref[0, :, :1]
    dv = lax.rsqrt(deg)
    t_ref[...] = hw_ref[...] * dv
    d_ref[...] = jnp.broadcast_to(dv, d_ref.shape)


def _layer2_body(s0, s1, dinv_ref, b1_ref, w2_ref, o_ref):
    h = jnp.concatenate([s0[...], s1[...]], axis=1)
    a = jnp.maximum(h * dinv_ref[:, :1] + b1_ref[...], 0.0)
    t = jnp.dot(a, w2_ref[0], preferred_element_type=jnp.float32)
    o_ref[...] = t * dinv_ref[:, :1]


def _make_pool_body(nb):
    def pool_body(s0, s1, dinv_ref, b2_ref, oh_ref,
                  o_ref, sums_ref, cnts_ref):
        i = pl.program_id(0)
        h = jnp.concatenate([s0[...], s1[...]], axis=1)
        out2 = h * dinv_ref[:, :1] + b2_ref[...]
        oh = oh_ref[...]
        ps = jax.lax.dot_general(oh, out2, (((0,), (0,)), ((), ())),
                                 preferred_element_type=jnp.float32)
        pc = jax.lax.dot_general(oh, jnp.ones_like(out2),
                                 (((0,), (0,)), ((), ())),
                                 preferred_element_type=jnp.float32)

        @pl.when(i == 0)
        def _():
            sums_ref[...] = ps
            cnts_ref[...] = pc

        @pl.when(i > 0)
        def _():
            sums_ref[...] += ps
            cnts_ref[...] += pc

        @pl.when(i == nb - 1)
        def _():
            o_ref[...] = sums_ref[...] / jnp.maximum(cnts_ref[...], 1.0)

    return pool_body


def kernel(x, edge_index, batch, W1, b1, W2, b2):
    N, D = x.shape
    E = edge_index.shape[1]
    Dh = D // 2
    nb = N // _BN

    src = edge_index[0]
    dst = edge_index[1]
    Epad = ((E + 16 * _CH - 1) // (16 * _CH)) * (16 * _CH)
    pad = Epad - E
    src_a = jnp.concatenate([src, jnp.zeros((pad,), jnp.int32)])
    src2 = jnp.concatenate(
        [src_a + q * N for q in range(2)]).reshape(2 * Epad // 128, 128)
    dst_p = jnp.concatenate(
        [dst, jnp.full((pad,), N, jnp.int32)]).reshape(Epad // 128, 128)

    zr = _acc_rows(N) // 16
    zeros_h = jnp.zeros((zr, Dh), jnp.float32)
    ones128 = jnp.ones((128, 128), jnp.float32)
    onehot = (batch[:, None] == jnp.arange(_G, dtype=batch.dtype)[None, :]
              ).astype(jnp.float32)
    b1_2d = b1.reshape(1, D)
    b2_2d = b2.reshape(1, D)
    W1s = W1.reshape(D, 2, Dh).transpose(1, 0, 2)
    W2s = W2.reshape(D, 2, Dh).transpose(1, 0, 2)

    degcount = _make_degcount(N, Epad)
    propagate = _make_propagate(N, Epad, Dh)

    hw = pl.pallas_call(
        _matmul_body,
        grid=(nb, 2),
        in_specs=[
            pl.BlockSpec((_BN, D), lambda i, j: (i, 0)),
            pl.BlockSpec((1, D, Dh), lambda i, j: (j, 0, 0)),
        ],
        out_specs=pl.BlockSpec((_BN, Dh), lambda i, j: (j * nb + i, 0)),
        out_shape=jax.ShapeDtypeStruct((2 * N, Dh), jnp.float32),
    )(x, W1s)

    cnt = degcount(dst_p, zeros_h, ones128)

    table1, dinv = pl.pallas_call(
        _scale_dinv_body,
        grid=(nb, 2),
        in_specs=[
            pl.BlockSpec((_BN, Dh), lambda i, j: (j * nb + i, 0)),
            pl.BlockSpec((1, _BN, 128), lambda i, j: (0, i, 0)),
            pl.BlockSpec((1, _BN, 128), lambda i, j: (1, i, 0)),
        ],
        out_specs=[
            pl.BlockSpec((_BN, Dh), lambda i, j: (j * nb + i, 0)),
            pl.BlockSpec((_BN, 128), lambda i, j: (i, 0)),
        ],
        out_shape=[
            jax.ShapeDtypeStruct((2 * N, Dh), jnp.float32),
            jax.ShapeDtypeStruct((N, 128), jnp.float32),
        ],
    )(hw, cnt, cnt)

    scat1 = propagate(table1, src2, dst_p)

    def _slab_spec(q):
        return pl.BlockSpec((_BN, Dh), lambda i, j, q=q: (q * nb + i, 0))

    table2 = pl.pallas_call(
        _layer2_body,
        grid=(nb, 2),
        in_specs=(
            [_slab_spec(q) for q in range(2)]
            + [
                pl.BlockSpec((_BN, 128), lambda i, j: (i, 0)),
                pl.BlockSpec((1, D), lambda i, j: (0, 0)),
                pl.BlockSpec((1, D, Dh), lambda i, j: (j, 0, 0)),
            ]
        ),
        out_specs=pl.BlockSpec((_BN, Dh), lambda i, j: (j * nb + i, 0)),
        out_shape=jax.ShapeDtypeStruct((2 * N, Dh), jnp.float32),
    )(scat1, scat1, dinv, b1_2d, W2s)

    scat2 = propagate(table2, src2, dst_p)

    def _slab_spec1(q):
        return pl.BlockSpec((_BN, Dh), lambda i, q=q: (q * nb + i, 0))

    graph_emb = pl.pallas_call(
        _make_pool_body(nb),
        grid=(nb,),
        in_specs=(
            [_slab_spec1(q) for q in range(2)]
            + [
                pl.BlockSpec((_BN, 128), lambda i: (i, 0)),
                pl.BlockSpec((1, D), lambda i: (0, 0)),
                pl.BlockSpec((_BN, _G), lambda i: (i, 0)),
            ]
        ),
        out_specs=pl.BlockSpec((_G, D), lambda i: (0, 0)),
        out_shape=jax.ShapeDtypeStruct((_G, D), jnp.float32),
        scratch_shapes=[
            pltpu.VMEM((_G, D), jnp.float32),
            pltpu.VMEM((_G, D), jnp.float32),
        ],
    )(scat2, scat2, dinv, b2_2d, onehot)

    return graph_emb


# speedup vs baseline: 1.0422x; 1.0396x over previous
import functools

import jax
import jax.numpy as jnp
from jax import lax
from jax.experimental import pallas as pl
from jax.experimental.pallas import tpu as pltpu
from jax.experimental.pallas import tpu_sc as plsc

_G = 8
_BN = 1000
_CH = 1024


def _acc_rows(N):
    return ((N + 1 + 127) // 128) * 128


def _make_degcount(N, Epad):
    acc_rows = _acc_rows(N)
    zr = acc_rows // 16
    rows_per_tile = Epad // 32 // 128
    mesh = plsc.VectorSubcoreMesh(core_axis_name="c", subcore_axis_name="s")

    @functools.partial(
        pl.kernel, mesh=mesh,
        out_type=jax.ShapeDtypeStruct((2, N, 128), jnp.float32),
        scratch_types=[
            pltpu.VMEM((8, 128), jnp.int32),
            pltpu.VMEM((128, 128), jnp.float32),
            pltpu.VMEM_SHARED((acc_rows, 128), jnp.float32),
            pltpu.SemaphoreType.DMA,
        ],
    )
    def degcount(dst_hbm, z16_hbm, ones_hbm, out_hbm, didx, ones_v, acc, sem):
        c = lax.axis_index("c")
        s = lax.axis_index("s")
        pltpu.sync_copy(ones_hbm, ones_v)
        pltpu.sync_copy(z16_hbm, acc.at[pl.ds(s * zr, zr)])
        plsc.subcore_barrier()

        row0 = (c * 16 + s) * rows_per_tile

        def chunk(i, carry):
            pltpu.sync_copy(dst_hbm.at[pl.ds(row0 + i * 8, 8)], didx)
            cps = [pltpu.async_copy(ones_v, acc.at[didx.at[j]], sem,
                                    add=True)
                   for j in range(8)]
            for cp in cps:
                cp.wait()
            return carry

        lax.fori_loop(0, rows_per_tile // 8, chunk, 0)
        plsc.subcore_barrier()
        npt = N // 10

        @pl.when(s < 10)
        def _():
            pltpu.sync_copy(acc.at[pl.ds(s * npt, npt)],
                            out_hbm.at[c, pl.ds(s * npt, npt)])

    return degcount


def _make_propagate(N, Epad, Dh):
    acc_rows = _acc_rows(N)
    zr = acc_rows // 16
    ept = Epad // 16
    ir = 40
    nck = ept // 128 // ir
    mesh = plsc.VectorSubcoreMesh(core_axis_name="c", subcore_axis_name="s")

    @functools.partial(
        pl.kernel, mesh=mesh,
        out_type=jax.ShapeDtypeStruct((2 * N, Dh), jnp.float32),
        scratch_types=[
            pltpu.VMEM((ir, 128), jnp.int32),
            pltpu.VMEM((ir, 128), jnp.int32),
            pltpu.VMEM((256, Dh), jnp.float32),
            pltpu.VMEM_SHARED((acc_rows, Dh), jnp.float32),
            pltpu.SemaphoreType.DMA,
            pltpu.SemaphoreType.DMA,
            pltpu.SemaphoreType.DMA,
            pltpu.SemaphoreType.DMA,
        ],
    )
    def propagate(table_hbm, src_hbm, dst_hbm, zer_hbm, out_hbm,
                  sidx, didx, rows, acc, sg0, sg1, ss0, ss1):
        c = lax.axis_index("c")
        s = lax.axis_index("s")
        dst_row0 = s * (ept // 128)
        src_row0 = c * (Epad // 128) + s * (ept // 128)
        npt = N // 10

        pltpu.sync_copy(zer_hbm, acc.at[pl.ds(s * zr, zr)])
        plsc.subcore_barrier()

        sg = (sg0, sg1)
        ss = (ss0, ss1)

        for cc in range(nck):
            sb, db = sidx, didx
            pltpu.sync_copy(src_hbm.at[pl.ds(src_row0 + cc * ir, ir)], sb)
            pltpu.sync_copy(dst_hbm.at[pl.ds(dst_row0 + cc * ir, ir)], db)

            def gth(k, sl, sb=sb):
                a = pltpu.async_copy(table_hbm.at[sb.at[k].at[pl.ds(0, 64)]],
                                     rows.at[pl.ds(sl * 128, 64)], sg[sl])
                b = pltpu.async_copy(table_hbm.at[sb.at[k].at[pl.ds(64, 64)]],
                                     rows.at[pl.ds(sl * 128 + 64, 64)],
                                     sg[sl])
                return (a, b)

            def sct(k, sl, db=db):
                return pltpu.async_copy(rows.at[pl.ds(sl * 128, 128)],
                                        acc.at[db.at[k]], ss[sl], add=True)

            def gwait(g):
                g[0].wait()
                g[1].wait()

            @pl.loop(0, ir, step=4)
            def _(k):
                g0 = gth(k, 0)
                g1 = gth(k + 1, 1)
                gwait(g0)
                s0 = sct(k, 0)
                gwait(g1)
                s1 = sct(k + 1, 1)
                s0.wait()
                g2 = gth(k + 2, 0)
                gwait(g2)
                s2 = sct(k + 2, 0)
                s1.wait()
                g3 = gth(k + 3, 1)
                gwait(g3)
                s3 = sct(k + 3, 1)
                s2.wait()
                s3.wait()

        plsc.subcore_barrier()

        @pl.when(s < 10)
        def _():
            pltpu.sync_copy(acc.at[pl.ds(s * npt, npt)],
                            out_hbm.at[pl.ds(c * N + s * npt, npt)])

    return propagate


def _matmul_body(x_ref, w_ref, o_ref):
    o_ref[...] = jnp.dot(x_ref[...], w_ref[0],
                         preferred_element_type=jnp.float32)


def _scale_dinv_body(hw_ref, a_ref, b_ref, t_ref, d_ref):
    deg = 1.0 + a_ref[0, :, :1] + b_ref[0, :, :1]
    dv = lax.rsqrt(deg)
    t_ref[...] = hw_ref[...] * dv
    d_ref[...] = jnp.broadcast_to(dv, d_ref.shape)


def _layer2_body(s0, s1, x0, x1, dinv_ref, b1_ref, w2_ref, o_ref):
    h = jnp.concatenate([s0[...] + x0[...], s1[...] + x1[...]], axis=1)
    a = jnp.maximum(h * dinv_ref[:, :1] + b1_ref[...], 0.0)
    t = jnp.dot(a, w2_ref[0], preferred_element_type=jnp.float32)
    o_ref[...] = t * dinv_ref[:, :1]


def _make_pool_body(nb):
    def pool_body(s0, s1, x0, x1, dinv_ref, b2_ref, oh_ref,
                  o_ref, sums_ref, cnts_ref):
        i = pl.program_id(0)
        h = jnp.concatenate([s0[...] + x0[...], s1[...] + x1[...]], axis=1)
        out2 = h * dinv_ref[:, :1] + b2_ref[...]
        oh = oh_ref[...]
        ps = jax.lax.dot_general(oh, out2, (((0,), (0,)), ((), ())),
                                 preferred_element_type=jnp.float32)
        pc = jax.lax.dot_general(oh, jnp.ones_like(out2),
                                 (((0,), (0,)), ((), ())),
                                 preferred_element_type=jnp.float32)

        @pl.when(i == 0)
        def _():
            sums_ref[...] = ps
            cnts_ref[...] = pc

        @pl.when(i > 0)
        def _():
            sums_ref[...] += ps
            cnts_ref[...] += pc

        @pl.when(i == nb - 1)
        def _():
            o_ref[...] = sums_ref[...] / jnp.maximum(cnts_ref[...], 1.0)

    return pool_body


def kernel(x, edge_index, batch, W1, b1, W2, b2):
    N, D = x.shape
    E = edge_index.shape[1]
    Dh = D // 2
    nb = N // _BN

    src = edge_index[0]
    dst = edge_index[1]
    Epad = ((E + 16 * _CH - 1) // (16 * _CH)) * (16 * _CH)
    pad = Epad - E
    src_a = jnp.concatenate([src, jnp.zeros((pad,), jnp.int32)])
    src2 = jnp.concatenate(
        [src_a + q * N for q in range(2)]).reshape(2 * Epad // 128, 128)
    dst_p = jnp.concatenate(
        [dst, jnp.full((pad,), N, jnp.int32)]).reshape(Epad // 128, 128)

    zr = _acc_rows(N) // 16
    zeros_h = jnp.zeros((zr, Dh), jnp.float32)
    ones128 = jnp.ones((128, 128), jnp.float32)
    onehot = (batch[:, None] == jnp.arange(_G, dtype=batch.dtype)[None, :]
              ).astype(jnp.float32)
    b1_2d = b1.reshape(1, D)
    b2_2d = b2.reshape(1, D)
    W1s = W1.reshape(D, 2, Dh).transpose(1, 0, 2)
    W2s = W2.reshape(D, 2, Dh).transpose(1, 0, 2)

    degcount = _make_degcount(N, Epad)
    propagate = _make_propagate(N, Epad, Dh)

    hw = pl.pallas_call(
        _matmul_body,
        grid=(nb, 2),
        in_specs=[
            pl.BlockSpec((_BN, D), lambda i, j: (i, 0)),
            pl.BlockSpec((1, D, Dh), lambda i, j: (j, 0, 0)),
        ],
        out_specs=pl.BlockSpec((_BN, Dh), lambda i, j: (j * nb + i, 0)),
        out_shape=jax.ShapeDtypeStruct((2 * N, Dh), jnp.float32),
    )(x, W1s)

    cnt = degcount(dst_p, zeros_h, ones128)

    table1, dinv = pl.pallas_call(
        _scale_dinv_body,
        grid=(nb, 2),
        in_specs=[
            pl.BlockSpec((_BN, Dh), lambda i, j: (j * nb + i, 0)),
            pl.BlockSpec((1, _BN, 128), lambda i, j: (0, i, 0)),
            pl.BlockSpec((1, _BN, 128), lambda i, j: (1, i, 0)),
        ],
        out_specs=[
            pl.BlockSpec((_BN, Dh), lambda i, j: (j * nb + i, 0)),
            pl.BlockSpec((_BN, 128), lambda i, j: (i, 0)),
        ],
        out_shape=[
            jax.ShapeDtypeStruct((2 * N, Dh), jnp.float32),
            jax.ShapeDtypeStruct((N, 128), jnp.float32),
        ],
    )(hw, cnt, cnt)

    scat1 = propagate(table1, src2, dst_p, zeros_h)

    def _slab_spec(q):
        return pl.BlockSpec((_BN, Dh), lambda i, j, q=q: (q * nb + i, 0))

    table2 = pl.pallas_call(
        _layer2_body,
        grid=(nb, 2),
        in_specs=(
            [_slab_spec(q) for q in range(2)]
            + [_slab_spec(q) for q in range(2)]
            + [
                pl.BlockSpec((_BN, 128), lambda i, j: (i, 0)),
                pl.BlockSpec((1, D), lambda i, j: (0, 0)),
                pl.BlockSpec((1, D, Dh), lambda i, j: (j, 0, 0)),
            ]
        ),
        out_specs=pl.BlockSpec((_BN, Dh), lambda i, j: (j * nb + i, 0)),
        out_shape=jax.ShapeDtypeStruct((2 * N, Dh), jnp.float32),
    )(scat1, scat1, table1, table1, dinv, b1_2d, W2s)

    scat2 = propagate(table2, src2, dst_p, zeros_h)

    def _slab_spec1(q):
        return pl.BlockSpec((_BN, Dh), lambda i, q=q: (q * nb + i, 0))

    graph_emb = pl.pallas_call(
        _make_pool_body(nb),
        grid=(nb,),
        in_specs=(
            [_slab_spec1(q) for q in range(2)]
            + [_slab_spec1(q) for q in range(2)]
            + [
                pl.BlockSpec((_BN, 128), lambda i: (i, 0)),
                pl.BlockSpec((1, D), lambda i: (0, 0)),
                pl.BlockSpec((_BN, _G), lambda i: (i, 0)),
            ]
        ),
        out_specs=pl.BlockSpec((_G, D), lambda i: (0, 0)),
        out_shape=jax.ShapeDtypeStruct((_G, D), jnp.float32),
        scratch_shapes=[
            pltpu.VMEM((_G, D), jnp.float32),
            pltpu.VMEM((_G, D), jnp.float32),
        ],
    )(scat2, scat2, table2, table2, dinv, b2_2d, onehot)

    return graph_emb


# speedup vs baseline: 1.0434x; 1.0012x over previous
import functools

import jax
import jax.numpy as jnp
from jax import lax
from jax.experimental import pallas as pl
from jax.experimental.pallas import tpu as pltpu
from jax.experimental.pallas import tpu_sc as plsc

_G = 8
_BN = 1000
_CH = 1024


def _acc_rows(N):
    return ((N + 1 + 127) // 128) * 128


def _make_degcount(N, Epad):
    acc_rows = _acc_rows(N)
    zr = acc_rows // 16
    rows_per_tile = Epad // 32 // 128
    mesh = plsc.VectorSubcoreMesh(core_axis_name="c", subcore_axis_name="s")

    @functools.partial(
        pl.kernel, mesh=mesh,
        out_type=jax.ShapeDtypeStruct((2, N, 128), jnp.float32),
        scratch_types=[
            pltpu.VMEM((8, 128), jnp.int32),
            pltpu.VMEM((128, 128), jnp.float32),
            pltpu.VMEM_SHARED((acc_rows, 128), jnp.float32),
            pltpu.SemaphoreType.DMA,
        ],
    )
    def degcount(dst_hbm, z16_hbm, ones_hbm, out_hbm, didx, ones_v, acc, sem):
        c = lax.axis_index("c")
        s = lax.axis_index("s")
        pltpu.sync_copy(ones_hbm, ones_v)
        pltpu.sync_copy(z16_hbm, acc.at[pl.ds(s * zr, zr)])
        plsc.subcore_barrier()

        row0 = (c * 16 + s) * rows_per_tile

        def chunk(i, carry):
            pltpu.sync_copy(dst_hbm.at[pl.ds(row0 + i * 8, 8)], didx)
            cps = [pltpu.async_copy(ones_v, acc.at[didx.at[j]], sem,
                                    add=True)
                   for j in range(8)]
            for cp in cps:
                cp.wait()
            return carry

        lax.fori_loop(0, rows_per_tile // 8, chunk, 0)
        plsc.subcore_barrier()
        npt = N // 10

        @pl.when(s < 10)
        def _():
            pltpu.sync_copy(acc.at[pl.ds(s * npt, npt)],
                            out_hbm.at[c, pl.ds(s * npt, npt)])

    return degcount


def _make_propagate(N, Epad, Dh):
    acc_rows = _acc_rows(N)
    zr = acc_rows // 16
    ept = Epad // 16
    ir = 40
    nck = ept // 128 // ir
    mesh = plsc.VectorSubcoreMesh(core_axis_name="c", subcore_axis_name="s")

    @functools.partial(
        pl.kernel, mesh=mesh,
        out_type=jax.ShapeDtypeStruct((2 * N, Dh), jnp.float32),
        scratch_types=[
            pltpu.VMEM((ir, 128), jnp.int32),
            pltpu.VMEM((ir, 128), jnp.int32),
            pltpu.VMEM((256, Dh), jnp.float32),
            pltpu.VMEM_SHARED((acc_rows, Dh), jnp.float32),
            pltpu.SemaphoreType.DMA,
            pltpu.SemaphoreType.DMA,
            pltpu.SemaphoreType.DMA,
            pltpu.SemaphoreType.DMA,
        ],
    )
    def propagate(table_hbm, src_hbm, dst_hbm, zer_hbm, out_hbm,
                  sidx, didx, rows, acc, sg0, sg1, ss0, ss1):
        c = lax.axis_index("c")
        s = lax.axis_index("s")
        dst_row0 = s * (ept // 128)
        src_row0 = c * (Epad // 128) + s * (ept // 128)
        npt = N // 10

        pltpu.sync_copy(zer_hbm, acc.at[pl.ds(s * zr, zr)])
        plsc.subcore_barrier()

        sg = (sg0, sg1)
        ss = (ss0, ss1)

        for cc in range(nck):
            sb, db = sidx, didx
            pltpu.sync_copy(src_hbm.at[pl.ds(src_row0 + cc * ir, ir)], sb)
            pltpu.sync_copy(dst_hbm.at[pl.ds(dst_row0 + cc * ir, ir)], db)

            def gth(k, sl, sb=sb):
                a = pltpu.async_copy(table_hbm.at[sb.at[k].at[pl.ds(0, 64)]],
                                     rows.at[pl.ds(sl * 128, 64)], sg[sl])
                b = pltpu.async_copy(table_hbm.at[sb.at[k].at[pl.ds(64, 64)]],
                                     rows.at[pl.ds(sl * 128 + 64, 64)],
                                     sg[sl])
                return (a, b)

            def sct(k, sl, db=db):
                return pltpu.async_copy(rows.at[pl.ds(sl * 128, 128)],
                                        acc.at[db.at[k]], ss[sl], add=True)

            def gwait(g):
                g[0].wait()
                g[1].wait()

            @pl.loop(0, ir, step=4)
            def _(k):
                g0 = gth(k, 0)
                g1 = gth(k + 1, 1)
                gwait(g0)
                s0 = sct(k, 0)
                gwait(g1)
                s1 = sct(k + 1, 1)
                s0.wait()
                g2 = gth(k + 2, 0)
                gwait(g2)
                s2 = sct(k + 2, 0)
                s1.wait()
                g3 = gth(k + 3, 1)
                gwait(g3)
                s3 = sct(k + 3, 1)
                s2.wait()
                s3.wait()

        plsc.subcore_barrier()

        @pl.when(s < 10)
        def _():
            pltpu.sync_copy(acc.at[pl.ds(s * npt, npt)],
                            out_hbm.at[pl.ds(c * N + s * npt, npt)])

    return propagate


def _matmul_body(x_ref, w_ref, o_ref):
    o_ref[...] = jnp.dot(x_ref[...], w_ref[0],
                         preferred_element_type=jnp.float32)


def _scale_dinv_body(hw_ref, a_ref, b_ref, t_ref, d_ref):
    deg = 1.0 + a_ref[0, :, :1] + b_ref[0, :, :1]
    dv = lax.rsqrt(deg)
    t_ref[...] = hw_ref[...] * dv
    d_ref[...] = jnp.broadcast_to(dv, d_ref.shape)


def _layer2_body(s0, s1, x0, x1, dinv_ref, b1_ref, w2_ref, o_ref):
    h = jnp.concatenate([s0[...] + x0[...], s1[...] + x1[...]], axis=1)
    a = jnp.maximum(h * dinv_ref[:, :1] + b1_ref[...], 0.0)
    t = jnp.dot(a, w2_ref[0], preferred_element_type=jnp.float32)
    o_ref[...] = t * dinv_ref[:, :1]


def _make_pool_body(nb):
    def pool_body(s0, s1, x0, x1, dinv_ref, b2_ref, oh_ref,
                  o_ref, sums_ref, cnts_ref):
        i = pl.program_id(0)
        h = jnp.concatenate([s0[...] + x0[...], s1[...] + x1[...]], axis=1)
        out2 = h * dinv_ref[:, :1] + b2_ref[...]
        oh = oh_ref[...]
        ps = jax.lax.dot_general(oh, out2, (((0,), (0,)), ((), ())),
                                 preferred_element_type=jnp.float32)
        pc = jax.lax.dot_general(oh, jnp.ones_like(out2),
                                 (((0,), (0,)), ((), ())),
                                 preferred_element_type=jnp.float32)

        @pl.when(i == 0)
        def _():
            sums_ref[...] = ps
            cnts_ref[...] = pc

        @pl.when(i > 0)
        def _():
            sums_ref[...] += ps
            cnts_ref[...] += pc

        @pl.when(i == nb - 1)
        def _():
            o_ref[...] = sums_ref[...] / jnp.maximum(cnts_ref[...], 1.0)

    return pool_body


def kernel(x, edge_index, batch, W1, b1, W2, b2):
    N, D = x.shape
    E = edge_index.shape[1]
    Dh = D // 2
    nb = N // _BN

    src = edge_index[0]
    dst = edge_index[1]
    Epad = ((E + 16 * _CH - 1) // (16 * _CH)) * (16 * _CH)
    pad = Epad - E
    src_a = jnp.concatenate([src, jnp.zeros((pad,), jnp.int32)])
    src2 = jnp.concatenate(
        [src_a + q * N for q in range(2)]).reshape(2 * Epad // 128, 128)
    dst_p = jnp.concatenate(
        [dst, jnp.full((pad,), N, jnp.int32)]).reshape(Epad // 128, 128)

    zr = _acc_rows(N) // 16
    zeros_h = jnp.zeros((zr, Dh), jnp.float32)
    ones128 = jnp.ones((128, 128), jnp.float32)
    onehot = (batch[:, None] == jnp.arange(_G, dtype=batch.dtype)[None, :]
              ).astype(jnp.float32)
    b1_2d = b1.reshape(1, D)
    b2_2d = b2.reshape(1, D)
    W1s = W1.reshape(D, 2, Dh).transpose(1, 0, 2)
    W2s = W2.reshape(D, 2, Dh).transpose(1, 0, 2)

    degcount = _make_degcount(N, Epad)
    propagate = _make_propagate(N, Epad, Dh)

    cnt = degcount(dst_p, zeros_h, ones128)

    hw = pl.pallas_call(
        _matmul_body,
        grid=(nb, 2),
        in_specs=[
            pl.BlockSpec((_BN, D), lambda i, j: (i, 0)),
            pl.BlockSpec((1, D, Dh), lambda i, j: (j, 0, 0)),
        ],
        out_specs=pl.BlockSpec((_BN, Dh), lambda i, j: (j * nb + i, 0)),
        out_shape=jax.ShapeDtypeStruct((2 * N, Dh), jnp.float32),
    )(x, W1s)

    table1, dinv = pl.pallas_call(
        _scale_dinv_body,
        grid=(nb, 2),
        in_specs=[
            pl.BlockSpec((_BN, Dh), lambda i, j: (j * nb + i, 0)),
            pl.BlockSpec((1, _BN, 128), lambda i, j: (0, i, 0)),
            pl.BlockSpec((1, _BN, 128), lambda i, j: (1, i, 0)),
        ],
        out_specs=[
            pl.BlockSpec((_BN, Dh), lambda i, j: (j * nb + i, 0)),
            pl.BlockSpec((_BN, 8), lambda i, j: (i, 0)),
        ],
        out_shape=[
            jax.ShapeDtypeStruct((2 * N, Dh), jnp.float32),
            jax.ShapeDtypeStruct((N, 8), jnp.float32),
        ],
    )(hw, cnt, cnt)

    scat1 = propagate(table1, src2, dst_p, zeros_h)

    def _slab_spec(q):
        return pl.BlockSpec((_BN, Dh), lambda i, j, q=q: (q * nb + i, 0))

    table2 = pl.pallas_call(
        _layer2_body,
        grid=(nb, 2),
        in_specs=(
            [_slab_spec(q) for q in range(2)]
            + [_slab_spec(q) for q in range(2)]
            + [
                pl.BlockSpec((_BN, 8), lambda i, j: (i, 0)),
                pl.BlockSpec((1, D), lambda i, j: (0, 0)),
                pl.BlockSpec((1, D, Dh), lambda i, j: (j, 0, 0)),
            ]
        ),
        out_specs=pl.BlockSpec((_BN, Dh), lambda i, j: (j * nb + i, 0)),
        out_shape=jax.ShapeDtypeStruct((2 * N, Dh), jnp.float32),
    )(scat1, scat1, table1, table1, dinv, b1_2d, W2s)

    scat2 = propagate(table2, src2, dst_p, zeros_h)

    def _slab_spec1(q):
        return pl.BlockSpec((_BN, Dh), lambda i, q=q: (q * nb + i, 0))

    graph_emb = pl.pallas_call(
        _make_pool_body(nb),
        grid=(nb,),
        in_specs=(
            [_slab_spec1(q) for q in range(2)]
            + [_slab_spec1(q) for q in range(2)]
            + [
                pl.BlockSpec((_BN, 8), lambda i: (i, 0)),
                pl.BlockSpec((1, D), lambda i: (0, 0)),
                pl.BlockSpec((_BN, _G), lambda i: (i, 0)),
            ]
        ),
        out_specs=pl.BlockSpec((_G, D), lambda i: (0, 0)),
        out_shape=jax.ShapeDtypeStruct((_G, D), jnp.float32),
        scratch_shapes=[
            pltpu.VMEM((_G, D), jnp.float32),
            pltpu.VMEM((_G, D), jnp.float32),
        ],
    )(scat2, scat2, table2, table2, dinv, b2_2d, onehot)

    return graph_emb
